# baseline (speedup 1.0000x reference)
import functools

import jax
import jax.numpy as jnp
from jax import lax
from jax.experimental import pallas as pl
from jax.experimental.pallas import tpu as pltpu
from jax.experimental.pallas import tpu_sc as plsc

F32 = jnp.float32

NC = 2
NS = 16
NW = NC * NS
RCH = 256


def _sc_mesh():
    return plsc.VectorSubcoreMesh(core_axis_name="c", subcore_axis_name="s")


def _sc_params():
    return pltpu.CompilerParams(
        use_tc_tiling_on_sc=False, needs_layout_passes=False)


def _zero_vmem(ref, nrows, d):
    zcols = d // 16

    def zrow(i, _):
        for k in range(zcols):
            ref[i, pl.ds(k * 16, 16)] = jnp.zeros((16,), F32)
        return _

    lax.fori_loop(0, nrows, zrow, None)


def _make_deg_kernel(n_pad, ecp):
    sl = n_pad // NS

    @functools.partial(
        pl.kernel,
        out_type=jax.ShapeDtypeStruct((NC * n_pad,), F32),
        mesh=_sc_mesh(),
        compiler_params=_sc_params(),
        scratch_types=[
            pltpu.VMEM((ecp,), jnp.int32),
            pltpu.VMEM((ecp,), F32),
            pltpu.VMEM((1, sl), F32),
            pltpu.VMEM_SHARED((n_pad,), F32),
        ],
    )
    def deg_kernel(dst_hbm, out_hbm, dst_v, ones_v, stage_v, acc):
        c = lax.axis_index("c")
        s = lax.axis_index("s")
        wid = c * NS + s

        def oinit(i, _):
            ones_v[pl.ds(i * 16, 16)] = jnp.ones((16,), F32)
            return _

        lax.fori_loop(0, ecp // 16, oinit, None)
        _zero_vmem(stage_v, 1, sl)
        pltpu.sync_copy(stage_v.at[0], acc.at[pl.ds(s * sl, sl)])
        plsc.subcore_barrier()

        pltpu.sync_copy(dst_hbm.at[wid], dst_v)
        pltpu.sync_copy(ones_v, acc.at[dst_v], add=True)
        plsc.subcore_barrier()

        pltpu.sync_copy(acc.at[pl.ds(s * sl, sl)], stage_v.at[0])
        pltpu.sync_copy(stage_v.at[0], out_hbm.at[pl.ds(c * n_pad + s * sl, sl)])

    return deg_kernel


def _make_agg_kernel(n_pad, ecp, d):
    sl = n_pad // NS
    nch = ecp // RCH
    dk = d // 16

    @functools.partial(
        pl.kernel,
        out_type=jax.ShapeDtypeStruct((NC, d, n_pad), F32),
        mesh=_sc_mesh(),
        compiler_params=_sc_params(),
        scratch_types=[
            pltpu.VMEM((ecp,), jnp.int32),
            pltpu.VMEM((ecp,), jnp.int32),
            pltpu.VMEM((2, RCH, d), F32),
            pltpu.VMEM((2, d * RCH), F32),
            pltpu.VMEM((d, sl), F32),
            pltpu.VMEM_SHARED((d, n_pad), F32),
            pltpu.SemaphoreType.DMA,
            pltpu.SemaphoreType.DMA,
        ],
    )
    def agg_kernel(g_hbm, src_hbm, dst_hbm, out_hbm,
                   src_v, dst_v, rows_v, col_v, stage_v, accT, gsem, ssem):
        c = lax.axis_index("c")
        s = lax.axis_index("s")
        wid = c * NS + s

        _zero_vmem(stage_v, d, sl)
        pltpu.sync_copy(stage_v, accT.at[:, pl.ds(s * sl, sl)])
        plsc.subcore_barrier()

        pltpu.sync_copy(src_hbm.at[wid], src_v)
        pltpu.sync_copy(dst_hbm.at[wid], dst_v)

        lanes = lax.iota(jnp.int32, 16)

        def transpose_row(b, r, _):
            for k in range(dk):
                v = rows_v[b, r, pl.ds(k * 16, 16)]
                idx = (lanes + (k * 16)) * RCH + r
                plsc.store_scatter(col_v.at[b], [idx], v)
            return _

        scat = [[None] * d, [None] * d]
        for i in range(nch):
            b = i & 1
            if scat[b][0] is not None:
                for dsc in scat[b]:
                    dsc.wait()
            pltpu.async_copy(
                g_hbm.at[src_v.at[pl.ds(i * RCH, RCH)]], rows_v.at[b], gsem,
            ).wait()
            lax.fori_loop(
                0, RCH, functools.partial(transpose_row, b), None)
            dsl = dst_v.at[pl.ds(i * RCH, RCH)]
            for j in range(d):
                scat[b][j] = pltpu.async_copy(
                    col_v.at[b, pl.ds(j * RCH, RCH)],
                    accT.at[j].at[dsl], ssem, add=True)
        for blist in scat:
            if blist[0] is not None:
                for dsc in blist:
                    dsc.wait()
        plsc.subcore_barrier()

        pltpu.sync_copy(accT.at[:, pl.ds(s * sl, sl)], stage_v)
        pltpu.sync_copy(stage_v, out_hbm.at[c, :, pl.ds(s * sl, sl)])

    return agg_kernel


def _tc_layer1(x_pad, w1, deg2, n_pad):
    f = x_pad.shape[1]
    h = w1.shape[1]
    br = 2048
    grid = n_pad // br

    def body(x_ref, w_ref, deg_ref, g_ref, dinv_ref):
        dinv = lax.rsqrt(deg_ref[0] + deg_ref[1] + 1.0)[None, :]
        hm = jnp.dot(x_ref[...], w_ref[...], preferred_element_type=F32)
        g_ref[...] = hm * dinv.T
        dinv_ref[...] = dinv

    return pl.pallas_call(
        body,
        grid=(grid,),
        in_specs=[
            pl.BlockSpec((br, f), lambda i: (i, 0)),
            pl.BlockSpec((f, h), lambda i: (0, 0)),
            pl.BlockSpec((NC, br), lambda i: (0, i)),
        ],
        out_specs=[
            pl.BlockSpec((br, h), lambda i: (i, 0)),
            pl.BlockSpec((1, br), lambda i: (0, i)),
        ],
        out_shape=[
            jax.ShapeDtypeStruct((n_pad, h), F32),
            jax.ShapeDtypeStruct((1, n_pad), F32),
        ],
    )(x_pad, w1, deg2)


def _tc_layer2(p, g1, dinvT, b1r, w2, n_pad):
    h = g1.shape[1]
    o = w2.shape[1]
    br = 2048
    grid = n_pad // br

    def body(p_ref, g1_ref, dinv_ref, b1_ref, w2_ref, g2_ref):
        dinv_col = dinv_ref[...].T
        p_rows = (p_ref[0] + p_ref[1]).T
        pre = dinv_col * (p_rows + g1_ref[...]) + b1_ref[...]
        a1 = jnp.maximum(pre, 0.0)
        h2 = jnp.dot(a1, w2_ref[...], preferred_element_type=F32)
        g2_ref[...] = h2 * dinv_col

    return pl.pallas_call(
        body,
        grid=(grid,),
        in_specs=[
            pl.BlockSpec((NC, h, br), lambda i: (0, 0, i)),
            pl.BlockSpec((br, h), lambda i: (i, 0)),
            pl.BlockSpec((1, br), lambda i: (0, i)),
            pl.BlockSpec((1, h), lambda i: (0, 0)),
            pl.BlockSpec((h, o), lambda i: (0, 0)),
        ],
        out_specs=pl.BlockSpec((br, o), lambda i: (i, 0)),
        out_shape=jax.ShapeDtypeStruct((n_pad, o), F32),
    )(p, g1, dinvT, b1r, w2)


def _tc_final(q, g2, dinvT, b2r, n_pad):
    o = g2.shape[1]
    br = 2048
    grid = n_pad // br

    def body(q_ref, g2_ref, dinv_ref, b2_ref, out_ref):
        q_rows = (q_ref[0] + q_ref[1]).T
        out_ref[...] = (
            dinv_ref[...].T * (q_rows + g2_ref[...]) + b2_ref[...]
        )

    return pl.pallas_call(
        body,
        grid=(grid,),
        in_specs=[
            pl.BlockSpec((NC, o, br), lambda i: (0, 0, i)),
            pl.BlockSpec((br, o), lambda i: (i, 0)),
            pl.BlockSpec((1, br), lambda i: (0, i)),
            pl.BlockSpec((1, o), lambda i: (0, 0)),
        ],
        out_specs=pl.BlockSpec((br, o), lambda i: (i, 0)),
        out_shape=jax.ShapeDtypeStruct((n_pad, o), F32),
    )(q, g2, dinvT, b2r)


def kernel(x, edge_index, W1, b1, W2, b2):
    n, f = x.shape
    e = edge_index.shape[1]
    h = W1.shape[1]
    o = W2.shape[1]

    n_pad = ((n + 2048) // 2048) * 2048

    ec = e // NW
    ecp = ((ec + RCH - 1) // RCH) * RCH
    padn = ecp - ec

    src_t = edge_index[0].reshape(NW, ec)
    dst_t = edge_index[1].reshape(NW, ec)
    if padn:
        pad_idx = n + (jnp.arange(padn, dtype=jnp.int32) % (n_pad - n))
        pads = jnp.broadcast_to(pad_idx, (NW, padn))
        src_t = jnp.concatenate([src_t, pads], axis=1)
        dst_t = jnp.concatenate([dst_t, pads], axis=1)

    x_pad = jnp.pad(x, ((0, n_pad - n), (0, 0)))

    deg2 = _make_deg_kernel(n_pad, ecp)(dst_t).reshape(NC, n_pad)
    g1, dinvT = _tc_layer1(x_pad, W1, deg2, n_pad)
    p = _make_agg_kernel(n_pad, ecp, h)(g1, src_t, dst_t)
    g2 = _tc_layer2(p, g1, dinvT, b1.reshape(1, h), W2, n_pad)
    q = _make_agg_kernel(n_pad, ecp, o)(g2, src_t, dst_t)
    out = _tc_final(q, g2, dinvT, b2.reshape(1, o), n_pad)
    return out[:n]
